# baseline (speedup 1.0000x reference)
import functools

import jax
import jax.numpy as jnp
from jax import lax
from jax.experimental import pallas as pl
from jax.experimental.pallas import tpu as pltpu
from jax.experimental.pallas import tpu_sc as plsc

D = 64
SCALE = float(D) ** 0.5

R = 4096
T = 200
NC = 2
NS = 16
NW = NC * NS
RW = R // NW
V2 = 500000
NG = 4
NO = 2

_mesh = plsc.VectorSubcoreMesh(core_axis_name="c", subcore_axis_name="s")


@functools.partial(
    pl.kernel,
    mesh=_mesh,
    out_type=jax.ShapeDtypeStruct((R, T, D), jnp.float32),
    scratch_types=[
        pltpu.VMEM((T, RW), jnp.int32),
        pltpu.VMEM((NG, RW), jnp.int32),
        pltpu.VMEM((NG, RW, 128), jnp.float32),
        pltpu.VMEM((NO, RW, D), jnp.float32),
        pltpu.SemaphoreType.DMA((NG,)),
        pltpu.SemaphoreType.DMA((NO,)),
    ],
    compiler_params=pltpu.CompilerParams(
        use_tc_tiling_on_sc=True, needs_layout_passes=False
    ),
)
def _embed(idxt_hbm, table2_hbm, out_hbm, idx_v, i2r_v, rows_v, oc_v, gsem, wsem):
    wid = lax.axis_index("s") * NC + lax.axis_index("c")
    r0 = wid * RW

    pltpu.sync_copy(idxt_hbm.at[:, pl.ds(r0, RW)], idx_v)

    def prep_fire_gather(t, b):
        for g in range(RW // 16):
            sl = pl.ds(g * 16, 16)
            i2r_v[b, sl] = lax.shift_right_logical(idx_v[t, sl], 1)
        pltpu.async_copy(table2_hbm.at[i2r_v.at[b]], rows_v.at[b], gsem.at[b])

    def wait_gather(b):
        pltpu.make_async_copy(
            table2_hbm.at[i2r_v.at[b]], rows_v.at[b], gsem.at[b]
        ).wait()

    def fire_write(t, ob):
        pltpu.async_copy(oc_v.at[ob], out_hbm.at[pl.ds(r0, RW), t], wsem.at[ob])

    def wait_write(t, ob):
        pltpu.make_async_copy(
            oc_v.at[ob], out_hbm.at[pl.ds(r0, RW), t], wsem.at[ob]
        ).wait()

    def compute(t, b, ob):
        def jg_body(jg, carry):
            sl = pl.ds(jg * 16, 16)
            idx16 = idx_v[t, sl]
            s = jnp.where(idx16 != 0, SCALE, 0.0).astype(jnp.float32)
            par = (idx16 & 1) * D
            for j in range(16):
                sj = s[j]
                base = par[j]
                row = jg * 16 + j
                for c in range(D // 16):
                    oc_v[ob, row, pl.ds(c * 16, 16)] = (
                        rows_v[b, row, pl.ds(base + c * 16, 16)] * sj
                    )
            return carry

        lax.fori_loop(0, RW // 16, jg_body, 0, unroll=2)

    for b in range(NG):
        prep_fire_gather(b, b)

    def step(k, carry):
        for u in range(NG):
            t = k * NG + u
            b = u
            ob = u % NO
            wait_gather(b)

            @pl.when(t >= NO)
            def _():
                wait_write(t - NO, ob)

            compute(t, b, ob)
            fire_write(t, ob)

            @pl.when(t + NG < T)
            def _():
                prep_fire_gather(t + NG, b)

        return carry

    lax.fori_loop(0, T // NG, step, 0)
    wait_write(T - 2, 0)
    wait_write(T - 1, 1)


def kernel(input, lookup_table):
    idxt = input.astype(jnp.int32).T
    table2 = lookup_table.reshape(V2, 2 * D)
    return _embed(idxt, table2)
